# Initial kernel scaffold; baseline (speedup 1.0000x reference)
import jax
import jax.numpy as jnp
from jax.experimental import pallas as pl


def kernel(x_in, adj, idx, W1, b1, W2, b2, W3, b3, W4, b4):
    raise NotImplementedError("write your pallas kernel here")


# speedup vs baseline: 1.0494x; 1.0494x over previous
import functools

import jax
import jax.numpy as jnp
from jax import lax
from jax.experimental import pallas as pl
from jax.experimental.pallas import tpu as pltpu
from jax.experimental.pallas import tpu_sc as plsc

_N = 10000
_N_SEG = 64
_BM = 200

_TILES = 32
_CHUNK = 128
_CPT = 3
_ROWS_PER_TILE = _CHUNK * _CPT
_N_PAD = _TILES * _ROWS_PER_TILE


def _in_proj_body(x_ref, w_ref, o_ref):
    o_ref[...] = jnp.dot(x_ref[...], w_ref[...],
                         preferred_element_type=jnp.float32)


def _in_proj(x, w):
    n, d = x.shape
    ow = w.shape[1]
    return pl.pallas_call(
        _in_proj_body,
        grid=(n // _BM,),
        in_specs=[
            pl.BlockSpec((_BM, d), lambda i: (i, 0)),
            pl.BlockSpec((d, ow), lambda i: (0, 0)),
        ],
        out_specs=pl.BlockSpec((_BM, ow), lambda i: (i, 0)),
        out_shape=jax.ShapeDtypeStruct((n, ow), jnp.float32),
        compiler_params=pltpu.CompilerParams(
            dimension_semantics=("parallel",)),
    )(x, w)


def _layer_body(adj_ref, y_ref, b_ref, wn_ref, bn_ref, o_ref):
    acc = jnp.dot(adj_ref[...], y_ref[...],
                  preferred_element_type=jnp.float32)
    h = jnp.maximum(acc + b_ref[...], 0.0)
    o_ref[...] = jnp.dot(h, wn_ref[...],
                         preferred_element_type=jnp.float32) + bn_ref[...]


def _fused_layer(adj, y, b, wn, bn):
    n = adj.shape[0]
    kdim = y.shape[1]
    ow = wn.shape[1]
    return pl.pallas_call(
        _layer_body,
        grid=(n // _BM,),
        in_specs=[
            pl.BlockSpec((_BM, n), lambda i: (i, 0)),
            pl.BlockSpec((n, kdim), lambda i: (0, 0)),
            pl.BlockSpec((1, kdim), lambda i: (0, 0)),
            pl.BlockSpec((kdim, ow), lambda i: (0, 0)),
            pl.BlockSpec((1, ow), lambda i: (0, 0)),
        ],
        out_specs=pl.BlockSpec((_BM, ow), lambda i: (i, 0)),
        out_shape=jax.ShapeDtypeStruct((n, ow), jnp.float32),
        compiler_params=pltpu.CompilerParams(
            dimension_semantics=("parallel",)),
    )(adj, y, b, wn, bn)


def _make_seg_sum():
    mesh = plsc.VectorSubcoreMesh(core_axis_name="c", subcore_axis_name="s")

    @functools.partial(
        pl.kernel,
        mesh=mesh,
        out_type=jax.ShapeDtypeStruct((_TILES, _N_SEG, _N_SEG), jnp.float32),
        scratch_types=[
            pltpu.VMEM((_ROWS_PER_TILE, _N_SEG), jnp.float32),
            pltpu.VMEM((_CPT, _CHUNK), jnp.int32),
            pltpu.VMEM((_N_SEG, _N_SEG), jnp.float32),
        ],
    )
    def seg_sum(x_hbm, idx_hbm, out_hbm, rows_v, idx_v, acc_v):
        cid = lax.axis_index("c")
        sid = lax.axis_index("s")
        wid = sid * 2 + cid
        pltpu.sync_copy(
            x_hbm.at[pl.ds(wid * _ROWS_PER_TILE, _ROWS_PER_TILE)], rows_v)
        pltpu.sync_copy(idx_hbm.at[wid], idx_v)

        zero = jnp.zeros((16,), jnp.float32)
        for r in range(_N_SEG):
            for j in range(_N_SEG // 16):
                acc_v[r, pl.ds(j * 16, 16)] = zero

        for c in range(_CPT):
            for g in range(_CHUNK // 16):
                svec = idx_v[c, pl.ds(g * 16, 16)]
                for k in range(16):
                    s = svec[k]
                    r = c * _CHUNK + g * 16 + k
                    for j in range(_N_SEG // 16):
                        plsc.addupdate(acc_v.at[s, pl.ds(j * 16, 16)],
                                       rows_v[r, pl.ds(j * 16, 16)])

        pltpu.sync_copy(acc_v, out_hbm.at[wid])

    return seg_sum


_seg_sum = _make_seg_sum()


def _tail_body(p_ref, o_ref):
    p = jnp.sum(p_ref[...], axis=0)
    m = jnp.max(p, axis=1, keepdims=True)
    s = jnp.sum(jnp.exp(p - m), axis=1, keepdims=True)
    o_ref[...] = (p - m) - jnp.log(s)


def _tail(parts):
    return pl.pallas_call(
        _tail_body,
        in_specs=[pl.BlockSpec((_TILES, _N_SEG, _N_SEG),
                               lambda: (0, 0, 0))],
        out_specs=pl.BlockSpec((_N_SEG, _N_SEG), lambda: (0, 0)),
        out_shape=jax.ShapeDtypeStruct((_N_SEG, _N_SEG), jnp.float32),
    )(parts)


def kernel(x_in, adj, idx, W1, b1, W2, b2, W3, b3, W4, b4):
    f32 = jnp.float32
    zeros2 = jnp.zeros((1, W2.shape[1]), f32)
    zeros3 = jnp.zeros((1, W3.shape[1]), f32)
    y1 = _in_proj(x_in, W1)
    y2 = _fused_layer(adj, y1, b1.reshape(1, -1), W2, zeros2)
    y3 = _fused_layer(adj, y2, b2.reshape(1, -1), W3, zeros3)
    x4 = _fused_layer(adj, y3, b3.reshape(1, -1), W4, b4.reshape(1, -1))

    x4p = jnp.pad(x4, ((0, _N_PAD - _N), (0, 0)))
    idxp = jnp.pad(idx.astype(jnp.int32), (0, _N_PAD - _N))
    idxp = idxp.reshape(_TILES, _CPT, _CHUNK)
    parts = _seg_sum(x4p, idxp)
    return _tail(parts)


# speedup vs baseline: 1.1936x; 1.1374x over previous
import functools

import jax
import jax.numpy as jnp
from jax import lax
from jax.experimental import pallas as pl
from jax.experimental.pallas import tpu as pltpu
from jax.experimental.pallas import tpu_sc as plsc

_N = 10000
_N_SEG = 64
_BM = 400

_TILES = 32
_CHUNK = 128
_CPT = 3
_ROWS_PER_TILE = _CHUNK * _CPT
_N_PAD = _TILES * _ROWS_PER_TILE


def _in_proj_body(x_ref, w_ref, o_ref):
    acc = jnp.dot(x_ref[...], w_ref[...], preferred_element_type=jnp.float32)
    o_ref[...] = acc.astype(jnp.bfloat16)


def _in_proj(x, w):
    n, d = x.shape
    ow = w.shape[1]
    return pl.pallas_call(
        _in_proj_body,
        grid=(n // _BM,),
        in_specs=[
            pl.BlockSpec((_BM, d), lambda i: (i, 0)),
            pl.BlockSpec((d, ow), lambda i: (0, 0)),
        ],
        out_specs=pl.BlockSpec((_BM, ow), lambda i: (i, 0)),
        out_shape=jax.ShapeDtypeStruct((n, ow), jnp.bfloat16),
        compiler_params=pltpu.CompilerParams(
            dimension_semantics=("parallel",)),
    )(x, w)


def _layer1_body(adj_ref, y_ref, b_ref, wn_ref, o_ref, adj_bf_ref):
    a_bf = adj_ref[...].astype(jnp.bfloat16)
    adj_bf_ref[...] = a_bf
    acc = jnp.dot(a_bf, y_ref[...], preferred_element_type=jnp.float32)
    h = jnp.maximum(acc + b_ref[...], 0.0).astype(jnp.bfloat16)
    o_ref[...] = jnp.dot(h, wn_ref[...], preferred_element_type=jnp.float32
                         ).astype(jnp.bfloat16)


def _layer1(adj, y, b, wn):
    n = adj.shape[0]
    kdim = y.shape[1]
    ow = wn.shape[1]
    return pl.pallas_call(
        _layer1_body,
        grid=(n // _BM,),
        in_specs=[
            pl.BlockSpec((_BM, n), lambda i: (i, 0)),
            pl.BlockSpec((n, kdim), lambda i: (0, 0)),
            pl.BlockSpec((1, kdim), lambda i: (0, 0)),
            pl.BlockSpec((kdim, ow), lambda i: (0, 0)),
        ],
        out_specs=[
            pl.BlockSpec((_BM, ow), lambda i: (i, 0)),
            pl.BlockSpec((_BM, n), lambda i: (i, 0)),
        ],
        out_shape=[
            jax.ShapeDtypeStruct((n, ow), jnp.bfloat16),
            jax.ShapeDtypeStruct((n, n), jnp.bfloat16),
        ],
        compiler_params=pltpu.CompilerParams(
            dimension_semantics=("parallel",)),
    )(adj, y, b, wn)


def _layer_body(adj_ref, y_ref, b_ref, wn_ref, bn_ref, o_ref, *, out_f32):
    acc = jnp.dot(adj_ref[...], y_ref[...],
                  preferred_element_type=jnp.float32)
    h = jnp.maximum(acc + b_ref[...], 0.0).astype(jnp.bfloat16)
    r = jnp.dot(h, wn_ref[...],
                preferred_element_type=jnp.float32) + bn_ref[...]
    o_ref[...] = r if out_f32 else r.astype(jnp.bfloat16)


def _fused_layer(adj_bf, y, b, wn, bn, out_f32):
    n = adj_bf.shape[0]
    kdim = y.shape[1]
    ow = wn.shape[1]
    return pl.pallas_call(
        functools.partial(_layer_body, out_f32=out_f32),
        grid=(n // _BM,),
        in_specs=[
            pl.BlockSpec((_BM, n), lambda i: (i, 0)),
            pl.BlockSpec((n, kdim), lambda i: (0, 0)),
            pl.BlockSpec((1, kdim), lambda i: (0, 0)),
            pl.BlockSpec((kdim, ow), lambda i: (0, 0)),
            pl.BlockSpec((1, ow), lambda i: (0, 0)),
        ],
        out_specs=pl.BlockSpec((_BM, ow), lambda i: (i, 0)),
        out_shape=jax.ShapeDtypeStruct(
            (n, ow), jnp.float32 if out_f32 else jnp.bfloat16),
        compiler_params=pltpu.CompilerParams(
            dimension_semantics=("parallel",)),
    )(adj_bf, y, b, wn, bn)


def _make_seg_sum():
    mesh = plsc.VectorSubcoreMesh(core_axis_name="c", subcore_axis_name="s")

    @functools.partial(
        pl.kernel,
        mesh=mesh,
        out_type=jax.ShapeDtypeStruct((_TILES, _N_SEG, _N_SEG), jnp.float32),
        scratch_types=[
            pltpu.VMEM((_ROWS_PER_TILE, _N_SEG), jnp.float32),
            pltpu.VMEM((_CPT, _CHUNK), jnp.int32),
            pltpu.VMEM((_N_SEG, _N_SEG), jnp.float32),
        ],
    )
    def seg_sum(x_hbm, idx_hbm, out_hbm, rows_v, idx_v, acc_v):
        cid = lax.axis_index("c")
        sid = lax.axis_index("s")
        wid = sid * 2 + cid
        pltpu.sync_copy(
            x_hbm.at[pl.ds(wid * _ROWS_PER_TILE, _ROWS_PER_TILE)], rows_v)
        pltpu.sync_copy(idx_hbm.at[wid], idx_v)

        zero = jnp.zeros((16,), jnp.float32)
        for r in range(_N_SEG):
            for j in range(_N_SEG // 16):
                acc_v[r, pl.ds(j * 16, 16)] = zero

        for c in range(_CPT):
            for g in range(_CHUNK // 16):
                svec = idx_v[c, pl.ds(g * 16, 16)]
                for k in range(16):
                    s = svec[k]
                    r = c * _CHUNK + g * 16 + k
                    for j in range(_N_SEG // 16):
                        plsc.addupdate(acc_v.at[s, pl.ds(j * 16, 16)],
                                       rows_v[r, pl.ds(j * 16, 16)])

        pltpu.sync_copy(acc_v, out_hbm.at[wid])

    return seg_sum


_seg_sum = _make_seg_sum()


def _tail_body(p_ref, o_ref):
    p = jnp.sum(p_ref[...], axis=0)
    m = jnp.max(p, axis=1, keepdims=True)
    s = jnp.sum(jnp.exp(p - m), axis=1, keepdims=True)
    o_ref[...] = (p - m) - jnp.log(s)


def _tail(parts):
    return pl.pallas_call(
        _tail_body,
        in_specs=[pl.BlockSpec((_TILES, _N_SEG, _N_SEG),
                               lambda: (0, 0, 0))],
        out_specs=pl.BlockSpec((_N_SEG, _N_SEG), lambda: (0, 0)),
        out_shape=jax.ShapeDtypeStruct((_N_SEG, _N_SEG), jnp.float32),
    )(parts)


def kernel(x_in, adj, idx, W1, b1, W2, b2, W3, b3, W4, b4):
    bf16 = jnp.bfloat16
    y1 = _in_proj(x_in, W1)
    y2, adj_bf = _layer1(adj, y1, b1.reshape(1, -1), W2.astype(bf16))
    y3 = _fused_layer(adj_bf, y2, b2.reshape(1, -1), W3.astype(bf16),
                      jnp.zeros((1, W3.shape[1]), jnp.float32), False)
    x4 = _fused_layer(adj_bf, y3, b3.reshape(1, -1), W4.astype(bf16),
                      b4.reshape(1, -1), True)

    x4p = jnp.pad(x4, ((0, _N_PAD - _N), (0, 0)))
    idxp = jnp.pad(idx.astype(jnp.int32), (0, _N_PAD - _N))
    idxp = idxp.reshape(_TILES, _CPT, _CHUNK)
    parts = _seg_sum(x4p, idxp)
    return _tail(parts)


# speedup vs baseline: 1.2041x; 1.0088x over previous
import functools

import jax
import jax.numpy as jnp
from jax import lax
from jax.experimental import pallas as pl
from jax.experimental.pallas import tpu as pltpu
from jax.experimental.pallas import tpu_sc as plsc

_N = 10000
_N_SEG = 64
_BM = 400

_TILES = 32
_CHUNK = 128
_CPT = 3
_ROWS_PER_TILE = _CHUNK * _CPT
_N_PAD = _TILES * _ROWS_PER_TILE


def _in_proj_body(x_ref, w_ref, o_ref):
    acc = jnp.dot(x_ref[...], w_ref[...], preferred_element_type=jnp.float32)
    o_ref[...] = acc.astype(jnp.bfloat16)


def _in_proj(x, w):
    n, d = x.shape
    ow = w.shape[1]
    return pl.pallas_call(
        _in_proj_body,
        grid=(n // _BM,),
        in_specs=[
            pl.BlockSpec((_BM, d), lambda i: (i, 0)),
            pl.BlockSpec((d, ow), lambda i: (0, 0)),
        ],
        out_specs=pl.BlockSpec((_BM, ow), lambda i: (i, 0)),
        out_shape=jax.ShapeDtypeStruct((n, ow), jnp.bfloat16),
        compiler_params=pltpu.CompilerParams(
            dimension_semantics=("parallel",)),
    )(x, w)


def _layer1_body(adj_ref, y_ref, b_ref, wn_ref, o_ref, adj_q_ref):
    a = adj_ref[...]
    adj_q_ref[...] = jnp.round(a * 127.0).astype(jnp.int8)
    acc = jnp.dot(a.astype(jnp.bfloat16), y_ref[...],
                  preferred_element_type=jnp.float32)
    h = jnp.maximum(acc + b_ref[...], 0.0).astype(jnp.bfloat16)
    o_ref[...] = jnp.dot(h, wn_ref[...], preferred_element_type=jnp.float32
                         ).astype(jnp.bfloat16)


def _layer1(adj, y, b, wn):
    n = adj.shape[0]
    kdim = y.shape[1]
    ow = wn.shape[1]
    return pl.pallas_call(
        _layer1_body,
        grid=(n // _BM,),
        in_specs=[
            pl.BlockSpec((_BM, n), lambda i: (i, 0)),
            pl.BlockSpec((n, kdim), lambda i: (0, 0)),
            pl.BlockSpec((1, kdim), lambda i: (0, 0)),
            pl.BlockSpec((kdim, ow), lambda i: (0, 0)),
        ],
        out_specs=[
            pl.BlockSpec((_BM, ow), lambda i: (i, 0)),
            pl.BlockSpec((_BM, n), lambda i: (i, 0)),
        ],
        out_shape=[
            jax.ShapeDtypeStruct((n, ow), jnp.bfloat16),
            jax.ShapeDtypeStruct((n, n), jnp.int8),
        ],
        compiler_params=pltpu.CompilerParams(
            dimension_semantics=("parallel",)),
    )(adj, y, b, wn)


def _colmax_body(y_ref, o_ref):
    @pl.when(pl.program_id(0) == 0)
    def _():
        o_ref[...] = jnp.zeros_like(o_ref)

    m = jnp.max(jnp.abs(y_ref[...].astype(jnp.float32)), axis=0,
                keepdims=True)
    o_ref[...] = jnp.maximum(o_ref[...], m)


def _colmax(y):
    n, k = y.shape
    return pl.pallas_call(
        _colmax_body,
        grid=(n // _BM,),
        in_specs=[pl.BlockSpec((_BM, k), lambda i: (i, 0))],
        out_specs=pl.BlockSpec((1, k), lambda i: (0, 0)),
        out_shape=jax.ShapeDtypeStruct((1, k), jnp.float32),
        compiler_params=pltpu.CompilerParams(
            dimension_semantics=("arbitrary",)),
    )(y)


def _quant_body(y_ref, cm_ref, q_ref, dq_ref):
    cm = jnp.maximum(cm_ref[...], 1e-20)
    r = 127.0 / cm
    q = jnp.round(y_ref[...].astype(jnp.float32) * r)
    q_ref[...] = jnp.clip(q, -127.0, 127.0).astype(jnp.int8)
    dq_ref[...] = cm * (1.0 / (127.0 * 127.0))


def _quant(y, cm):
    n, k = y.shape
    return pl.pallas_call(
        _quant_body,
        grid=(n // _BM,),
        in_specs=[
            pl.BlockSpec((_BM, k), lambda i: (i, 0)),
            pl.BlockSpec((1, k), lambda i: (0, 0)),
        ],
        out_specs=[
            pl.BlockSpec((_BM, k), lambda i: (i, 0)),
            pl.BlockSpec((1, k), lambda i: (0, 0)),
        ],
        out_shape=[
            jax.ShapeDtypeStruct((n, k), jnp.int8),
            jax.ShapeDtypeStruct((1, k), jnp.float32),
        ],
        compiler_params=pltpu.CompilerParams(
            dimension_semantics=("arbitrary",)),
    )(y, cm)


def _layer_body(adj_ref, y_ref, dq_ref, b_ref, wn_ref, bn_ref, o_ref, *,
                out_f32):
    acc = jnp.dot(adj_ref[...], y_ref[...],
                  preferred_element_type=jnp.int32)
    accf = acc.astype(jnp.float32) * dq_ref[...]
    h = jnp.maximum(accf + b_ref[...], 0.0).astype(jnp.bfloat16)
    r = jnp.dot(h, wn_ref[...],
                preferred_element_type=jnp.float32) + bn_ref[...]
    o_ref[...] = r if out_f32 else r.astype(jnp.bfloat16)


def _fused_layer(adj_q, yq, dq, b, wn, bn, out_f32):
    n = adj_q.shape[0]
    kdim = yq.shape[1]
    ow = wn.shape[1]
    return pl.pallas_call(
        functools.partial(_layer_body, out_f32=out_f32),
        grid=(n // _BM,),
        in_specs=[
            pl.BlockSpec((_BM, n), lambda i: (i, 0)),
            pl.BlockSpec((n, kdim), lambda i: (0, 0)),
            pl.BlockSpec((1, kdim), lambda i: (0, 0)),
            pl.BlockSpec((1, kdim), lambda i: (0, 0)),
            pl.BlockSpec((kdim, ow), lambda i: (0, 0)),
            pl.BlockSpec((1, ow), lambda i: (0, 0)),
        ],
        out_specs=pl.BlockSpec((_BM, ow), lambda i: (i, 0)),
        out_shape=jax.ShapeDtypeStruct(
            (n, ow), jnp.float32 if out_f32 else jnp.bfloat16),
        compiler_params=pltpu.CompilerParams(
            dimension_semantics=("parallel",)),
    )(adj_q, yq, dq, b, wn, bn)


def _make_seg_sum():
    mesh = plsc.VectorSubcoreMesh(core_axis_name="c", subcore_axis_name="s")

    @functools.partial(
        pl.kernel,
        mesh=mesh,
        out_type=jax.ShapeDtypeStruct((_TILES, _N_SEG, _N_SEG), jnp.float32),
        scratch_types=[
            pltpu.VMEM((_ROWS_PER_TILE, _N_SEG), jnp.float32),
            pltpu.VMEM((_CPT, _CHUNK), jnp.int32),
            pltpu.VMEM((_N_SEG, _N_SEG), jnp.float32),
        ],
    )
    def seg_sum(x_hbm, idx_hbm, out_hbm, rows_v, idx_v, acc_v):
        cid = lax.axis_index("c")
        sid = lax.axis_index("s")
        wid = sid * 2 + cid
        pltpu.sync_copy(
            x_hbm.at[pl.ds(wid * _ROWS_PER_TILE, _ROWS_PER_TILE)], rows_v)
        pltpu.sync_copy(idx_hbm.at[wid], idx_v)

        zero = jnp.zeros((16,), jnp.float32)
        for r in range(_N_SEG):
            for j in range(_N_SEG // 16):
                acc_v[r, pl.ds(j * 16, 16)] = zero

        for c in range(_CPT):
            for g in range(_CHUNK // 16):
                svec = idx_v[c, pl.ds(g * 16, 16)]
                for k in range(16):
                    s = svec[k]
                    r = c * _CHUNK + g * 16 + k
                    for j in range(_N_SEG // 16):
                        plsc.addupdate(acc_v.at[s, pl.ds(j * 16, 16)],
                                       rows_v[r, pl.ds(j * 16, 16)])

        pltpu.sync_copy(acc_v, out_hbm.at[wid])

    return seg_sum


_seg_sum = _make_seg_sum()


def _tail_body(p_ref, o_ref):
    p = jnp.sum(p_ref[...], axis=0)
    m = jnp.max(p, axis=1, keepdims=True)
    s = jnp.sum(jnp.exp(p - m), axis=1, keepdims=True)
    o_ref[...] = (p - m) - jnp.log(s)


def _tail(parts):
    return pl.pallas_call(
        _tail_body,
        in_specs=[pl.BlockSpec((_TILES, _N_SEG, _N_SEG),
                               lambda: (0, 0, 0))],
        out_specs=pl.BlockSpec((_N_SEG, _N_SEG), lambda: (0, 0)),
        out_shape=jax.ShapeDtypeStruct((_N_SEG, _N_SEG), jnp.float32),
    )(parts)


def kernel(x_in, adj, idx, W1, b1, W2, b2, W3, b3, W4, b4):
    bf16 = jnp.bfloat16
    y1 = _in_proj(x_in, W1)
    y2, adj_q = _layer1(adj, y1, b1.reshape(1, -1), W2.astype(bf16))
    y2q, dq2 = _quant(y2, _colmax(y2))
    y3 = _fused_layer(adj_q, y2q, dq2, b2.reshape(1, -1), W3.astype(bf16),
                      jnp.zeros((1, W3.shape[1]), jnp.float32), False)
    y3q, dq3 = _quant(y3, _colmax(y3))
    x4 = _fused_layer(adj_q, y3q, dq3, b3.reshape(1, -1), W4.astype(bf16),
                      b4.reshape(1, -1), True)

    x4p = jnp.pad(x4, ((0, _N_PAD - _N), (0, 0)))
    idxp = jnp.pad(idx.astype(jnp.int32), (0, _N_PAD - _N))
    idxp = idxp.reshape(_TILES, _CPT, _CHUNK)
    parts = _seg_sum(x4p, idxp)
    return _tail(parts)


# speedup vs baseline: 1.4135x; 1.1739x over previous
import functools

import jax
import jax.numpy as jnp
from jax import lax
from jax.experimental import pallas as pl
from jax.experimental.pallas import tpu as pltpu
from jax.experimental.pallas import tpu_sc as plsc

_N = 10000
_N_SEG = 64
_BM = 400

_TILES = 32
_CHUNK = 128
_CPT = 3
_ROWS_PER_TILE = _CHUNK * _CPT
_N_PAD = _TILES * _ROWS_PER_TILE


def _in_proj_body(x_ref, w_ref, o_ref):
    acc = jnp.dot(x_ref[...], w_ref[...], preferred_element_type=jnp.float32)
    o_ref[...] = acc.astype(jnp.bfloat16)


def _in_proj(x, w):
    n, d = x.shape
    ow = w.shape[1]
    return pl.pallas_call(
        _in_proj_body,
        grid=(n // _BM,),
        in_specs=[
            pl.BlockSpec((_BM, d), lambda i: (i, 0)),
            pl.BlockSpec((d, ow), lambda i: (0, 0)),
        ],
        out_specs=pl.BlockSpec((_BM, ow), lambda i: (i, 0)),
        out_shape=jax.ShapeDtypeStruct((n, ow), jnp.bfloat16),
        compiler_params=pltpu.CompilerParams(
            dimension_semantics=("parallel",)),
    )(x, w)


def _layer1_body(adj_ref, y_ref, b_ref, wn_ref, o_ref, adj_q_ref):
    a = adj_ref[...]
    adj_q_ref[...] = jnp.round(a * 127.0).astype(jnp.int8)
    acc = jnp.dot(a.astype(jnp.bfloat16), y_ref[...],
                  preferred_element_type=jnp.float32)
    h = jnp.maximum(acc + b_ref[...], 0.0).astype(jnp.bfloat16)
    o_ref[...] = (jnp.dot(h, wn_ref[...], preferred_element_type=jnp.float32)
                  * (1.0 / 127.0)).astype(jnp.bfloat16)


def _layer1(adj, y, b, wn):
    n = adj.shape[0]
    kdim = y.shape[1]
    ow = wn.shape[1]
    return pl.pallas_call(
        _layer1_body,
        grid=(n // _BM,),
        in_specs=[
            pl.BlockSpec((_BM, n), lambda i: (i, 0)),
            pl.BlockSpec((n, kdim), lambda i: (0, 0)),
            pl.BlockSpec((1, kdim), lambda i: (0, 0)),
            pl.BlockSpec((kdim, ow), lambda i: (0, 0)),
        ],
        out_specs=[
            pl.BlockSpec((_BM, ow), lambda i: (i, 0)),
            pl.BlockSpec((_BM, n), lambda i: (i, 0)),
        ],
        out_shape=[
            jax.ShapeDtypeStruct((n, ow), jnp.bfloat16),
            jax.ShapeDtypeStruct((n, n), jnp.int8),
        ],
        compiler_params=pltpu.CompilerParams(
            dimension_semantics=("parallel",)),
    )(adj, y, b, wn)


_BM2 = 1000


def _layer_body(adj_ref, y_ref, b_ref, wn_ref, bn_ref, o_ref, *, out_f32):
    a_bf = adj_ref[...].astype(jnp.bfloat16)
    acc = jnp.dot(a_bf, y_ref[...], preferred_element_type=jnp.float32)
    h = jnp.maximum(acc + b_ref[...], 0.0).astype(jnp.bfloat16)
    r = jnp.dot(h, wn_ref[...], preferred_element_type=jnp.float32)
    if out_f32:
        o_ref[...] = r + bn_ref[...]
    else:
        o_ref[...] = (r * (1.0 / 127.0)).astype(jnp.bfloat16)


def _fused_layer(adj_q, y, b, wn, bn, out_f32):
    n = adj_q.shape[0]
    kdim = y.shape[1]
    ow = wn.shape[1]
    return pl.pallas_call(
        functools.partial(_layer_body, out_f32=out_f32),
        grid=(n // _BM2,),
        in_specs=[
            pl.BlockSpec((_BM2, n), lambda i: (i, 0)),
            pl.BlockSpec((n, kdim), lambda i: (0, 0)),
            pl.BlockSpec((1, kdim), lambda i: (0, 0)),
            pl.BlockSpec((kdim, ow), lambda i: (0, 0)),
            pl.BlockSpec((1, ow), lambda i: (0, 0)),
        ],
        out_specs=pl.BlockSpec((_BM2, ow), lambda i: (i, 0)),
        out_shape=jax.ShapeDtypeStruct(
            (n, ow), jnp.float32 if out_f32 else jnp.bfloat16),
        compiler_params=pltpu.CompilerParams(
            dimension_semantics=("parallel",)),
    )(adj_q, y, b, wn, bn)


def _make_seg_sum():
    mesh = plsc.VectorSubcoreMesh(core_axis_name="c", subcore_axis_name="s")

    @functools.partial(
        pl.kernel,
        mesh=mesh,
        out_type=jax.ShapeDtypeStruct((_TILES, _N_SEG, _N_SEG), jnp.float32),
        scratch_types=[
            pltpu.VMEM((_ROWS_PER_TILE, _N_SEG), jnp.float32),
            pltpu.VMEM((_CPT, _CHUNK), jnp.int32),
            pltpu.VMEM((_N_SEG, _N_SEG), jnp.float32),
        ],
    )
    def seg_sum(x_hbm, idx_hbm, out_hbm, rows_v, idx_v, acc_v):
        cid = lax.axis_index("c")
        sid = lax.axis_index("s")
        wid = sid * 2 + cid
        pltpu.sync_copy(
            x_hbm.at[pl.ds(wid * _ROWS_PER_TILE, _ROWS_PER_TILE)], rows_v)
        pltpu.sync_copy(idx_hbm.at[wid], idx_v)

        zero = jnp.zeros((16,), jnp.float32)
        for r in range(_N_SEG):
            for j in range(_N_SEG // 16):
                acc_v[r, pl.ds(j * 16, 16)] = zero

        for c in range(_CPT):
            for g in range(_CHUNK // 16):
                svec = idx_v[c, pl.ds(g * 16, 16)]
                for k in range(16):
                    s = svec[k]
                    r = c * _CHUNK + g * 16 + k
                    for j in range(_N_SEG // 16):
                        plsc.addupdate(acc_v.at[s, pl.ds(j * 16, 16)],
                                       rows_v[r, pl.ds(j * 16, 16)])

        pltpu.sync_copy(acc_v, out_hbm.at[wid])

    return seg_sum


_seg_sum = _make_seg_sum()


def _tail_body(p_ref, o_ref):
    p = jnp.sum(p_ref[...], axis=0)
    m = jnp.max(p, axis=1, keepdims=True)
    s = jnp.sum(jnp.exp(p - m), axis=1, keepdims=True)
    o_ref[...] = (p - m) - jnp.log(s)


def _tail(parts):
    return pl.pallas_call(
        _tail_body,
        in_specs=[pl.BlockSpec((_TILES, _N_SEG, _N_SEG),
                               lambda: (0, 0, 0))],
        out_specs=pl.BlockSpec((_N_SEG, _N_SEG), lambda: (0, 0)),
        out_shape=jax.ShapeDtypeStruct((_N_SEG, _N_SEG), jnp.float32),
    )(parts)


def kernel(x_in, adj, idx, W1, b1, W2, b2, W3, b3, W4, b4):
    bf16 = jnp.bfloat16
    y1 = _in_proj(x_in, W1)
    y2, adj_q = _layer1(adj, y1, b1.reshape(1, -1), W2.astype(bf16))
    y3 = _fused_layer(adj_q, y2, b2.reshape(1, -1), W3.astype(bf16),
                      jnp.zeros((1, W3.shape[1]), jnp.float32), False)
    x4 = _fused_layer(adj_q, y3, b3.reshape(1, -1), W4.astype(bf16),
                      b4.reshape(1, -1), True)

    x4p = jnp.pad(x4, ((0, _N_PAD - _N), (0, 0)))
    idxp = jnp.pad(idx.astype(jnp.int32), (0, _N_PAD - _N))
    idxp = idxp.reshape(_TILES, _CPT, _CHUNK)
    parts = _seg_sum(x4p, idxp)
    return _tail(parts)


# speedup vs baseline: 1.4702x; 1.0401x over previous
import functools

import jax
import jax.numpy as jnp
from jax import lax
from jax.experimental import pallas as pl
from jax.experimental.pallas import tpu as pltpu
from jax.experimental.pallas import tpu_sc as plsc

_N = 10000
_N_SEG = 64
_BM = 200

_TILES = 32
_CHUNK = 128
_CPT = 3
_ROWS_PER_TILE = _CHUNK * _CPT
_N_PAD = _TILES * _ROWS_PER_TILE


def _layer1_body(x_ref, w1_ref, adj_ref, b_ref, wn_ref, o_ref, adj_q_ref,
                 y1_ref):
    @pl.when(pl.program_id(0) == 0)
    def _():
        y1_ref[...] = jnp.dot(
            x_ref[...], w1_ref[...],
            preferred_element_type=jnp.float32).astype(jnp.bfloat16)

    a = adj_ref[...]
    adj_q_ref[...] = jnp.round(a * 127.0).astype(jnp.int8)
    acc = jnp.dot(a.astype(jnp.bfloat16), y1_ref[...],
                  preferred_element_type=jnp.float32)
    h = jnp.maximum(acc + b_ref[...], 0.0).astype(jnp.bfloat16)
    o_ref[...] = (jnp.dot(h, wn_ref[...], preferred_element_type=jnp.float32)
                  * (1.0 / 127.0)).astype(jnp.bfloat16)


def _layer1(x_in, W1, adj, b, wn):
    n = adj.shape[0]
    d = x_in.shape[1]
    kdim = W1.shape[1]
    ow = wn.shape[1]
    return pl.pallas_call(
        _layer1_body,
        grid=(n // _BM,),
        in_specs=[
            pl.BlockSpec((n, d), lambda i: (0, 0)),
            pl.BlockSpec((d, kdim), lambda i: (0, 0)),
            pl.BlockSpec((_BM, n), lambda i: (i, 0)),
            pl.BlockSpec((1, kdim), lambda i: (0, 0)),
            pl.BlockSpec((kdim, ow), lambda i: (0, 0)),
        ],
        out_specs=[
            pl.BlockSpec((_BM, ow), lambda i: (i, 0)),
            pl.BlockSpec((_BM, n), lambda i: (i, 0)),
        ],
        out_shape=[
            jax.ShapeDtypeStruct((n, ow), jnp.bfloat16),
            jax.ShapeDtypeStruct((n, n), jnp.int8),
        ],
        scratch_shapes=[pltpu.VMEM((n, kdim), jnp.bfloat16)],
        compiler_params=pltpu.CompilerParams(
            dimension_semantics=("arbitrary",)),
    )(x_in, W1, adj, b, wn)


_BM2 = 1000


def _layer_body(adj_ref, y_ref, b_ref, wn_ref, bn_ref, o_ref, *, out_f32):
    a_bf = adj_ref[...].astype(jnp.bfloat16)
    acc = jnp.dot(a_bf, y_ref[...], preferred_element_type=jnp.float32)
    h = jnp.maximum(acc + b_ref[...], 0.0).astype(jnp.bfloat16)
    r = jnp.dot(h, wn_ref[...], preferred_element_type=jnp.float32)
    if out_f32:
        o_ref[...] = r + bn_ref[...]
    else:
        o_ref[...] = (r * (1.0 / 127.0)).astype(jnp.bfloat16)


def _fused_layer(adj_q, y, b, wn, bn, out_f32):
    n = adj_q.shape[0]
    kdim = y.shape[1]
    ow = wn.shape[1]
    return pl.pallas_call(
        functools.partial(_layer_body, out_f32=out_f32),
        grid=(n // _BM2,),
        in_specs=[
            pl.BlockSpec((_BM2, n), lambda i: (i, 0)),
            pl.BlockSpec((n, kdim), lambda i: (0, 0)),
            pl.BlockSpec((1, kdim), lambda i: (0, 0)),
            pl.BlockSpec((kdim, ow), lambda i: (0, 0)),
            pl.BlockSpec((1, ow), lambda i: (0, 0)),
        ],
        out_specs=pl.BlockSpec((_BM2, ow), lambda i: (i, 0)),
        out_shape=jax.ShapeDtypeStruct(
            (n, ow), jnp.float32 if out_f32 else jnp.bfloat16),
        compiler_params=pltpu.CompilerParams(
            dimension_semantics=("parallel",)),
    )(adj_q, y, b, wn, bn)


def _make_seg_sum():
    mesh = plsc.VectorSubcoreMesh(core_axis_name="c", subcore_axis_name="s")

    @functools.partial(
        pl.kernel,
        mesh=mesh,
        out_type=jax.ShapeDtypeStruct((_TILES, _N_SEG, _N_SEG), jnp.float32),
        scratch_types=[
            pltpu.VMEM((_ROWS_PER_TILE, _N_SEG), jnp.float32),
            pltpu.VMEM((_CPT, _CHUNK), jnp.int32),
            pltpu.VMEM((_N_SEG, _N_SEG), jnp.float32),
        ],
    )
    def seg_sum(x_hbm, idx_hbm, out_hbm, rows_v, idx_v, acc_v):
        cid = lax.axis_index("c")
        sid = lax.axis_index("s")
        wid = sid * 2 + cid
        pltpu.sync_copy(
            x_hbm.at[pl.ds(wid * _ROWS_PER_TILE, _ROWS_PER_TILE)], rows_v)
        pltpu.sync_copy(idx_hbm.at[wid], idx_v)

        zero = jnp.zeros((16,), jnp.float32)
        for r in range(_N_SEG):
            for j in range(_N_SEG // 16):
                acc_v[r, pl.ds(j * 16, 16)] = zero

        for c in range(_CPT):
            for g in range(_CHUNK // 16):
                svec = idx_v[c, pl.ds(g * 16, 16)]
                for k in range(16):
                    s = svec[k]
                    r = c * _CHUNK + g * 16 + k
                    for j in range(_N_SEG // 16):
                        plsc.addupdate(acc_v.at[s, pl.ds(j * 16, 16)],
                                       rows_v[r, pl.ds(j * 16, 16)])

        pltpu.sync_copy(acc_v, out_hbm.at[wid])

    return seg_sum


_seg_sum = _make_seg_sum()


def _tail_body(p_ref, o_ref):
    p = jnp.sum(p_ref[...], axis=0)
    m = jnp.max(p, axis=1, keepdims=True)
    s = jnp.sum(jnp.exp(p - m), axis=1, keepdims=True)
    o_ref[...] = (p - m) - jnp.log(s)


def _tail(parts):
    return pl.pallas_call(
        _tail_body,
        in_specs=[pl.BlockSpec((_TILES, _N_SEG, _N_SEG),
                               lambda: (0, 0, 0))],
        out_specs=pl.BlockSpec((_N_SEG, _N_SEG), lambda: (0, 0)),
        out_shape=jax.ShapeDtypeStruct((_N_SEG, _N_SEG), jnp.float32),
    )(parts)


def kernel(x_in, adj, idx, W1, b1, W2, b2, W3, b3, W4, b4):
    bf16 = jnp.bfloat16
    y2, adj_q = _layer1(x_in, W1, adj, b1.reshape(1, -1), W2.astype(bf16))
    y3 = _fused_layer(adj_q, y2, b2.reshape(1, -1), W3.astype(bf16),
                      jnp.zeros((1, W3.shape[1]), jnp.float32), False)
    x4 = _fused_layer(adj_q, y3, b3.reshape(1, -1), W4.astype(bf16),
                      b4.reshape(1, -1), True)

    x4p = jnp.pad(x4, ((0, _N_PAD - _N), (0, 0)))
    idxp = jnp.pad(idx.astype(jnp.int32), (0, _N_PAD - _N))
    idxp = idxp.reshape(_TILES, _CPT, _CHUNK)
    parts = _seg_sum(x4p, idxp)
    return _tail(parts)


# speedup vs baseline: 1.4795x; 1.0063x over previous
import functools

import jax
import jax.numpy as jnp
from jax import lax
from jax.experimental import pallas as pl
from jax.experimental.pallas import tpu as pltpu
from jax.experimental.pallas import tpu_sc as plsc

_N = 10000
_N_SEG = 64
_BM = 200

_TILES = 32
_CHUNK = 128
_CPT = 3
_ROWS_PER_TILE = _CHUNK * _CPT
_N_PAD = _TILES * _ROWS_PER_TILE


def _layer1_body(x_ref, w1_ref, adj_ref, b_ref, wn_ref, o_ref, adj_q_ref,
                 y1_ref):
    @pl.when(pl.program_id(0) == 0)
    def _():
        y1_ref[...] = jnp.dot(
            x_ref[...], w1_ref[...],
            preferred_element_type=jnp.float32).astype(jnp.bfloat16)

    a = adj_ref[...]
    adj_q_ref[...] = jnp.round(a * 127.0).astype(jnp.int8)
    acc = jnp.dot(a.astype(jnp.bfloat16), y1_ref[...],
                  preferred_element_type=jnp.float32)
    h = jnp.maximum(acc + b_ref[...], 0.0).astype(jnp.bfloat16)
    o_ref[...] = (jnp.dot(h, wn_ref[...], preferred_element_type=jnp.float32)
                  * (1.0 / 127.0)).astype(jnp.bfloat16)


def _layer1(x_in, W1, adj, b, wn):
    n = adj.shape[0]
    d = x_in.shape[1]
    kdim = W1.shape[1]
    ow = wn.shape[1]
    return pl.pallas_call(
        _layer1_body,
        grid=(n // _BM,),
        in_specs=[
            pl.BlockSpec((n, d), lambda i: (0, 0)),
            pl.BlockSpec((d, kdim), lambda i: (0, 0)),
            pl.BlockSpec((_BM, n), lambda i: (i, 0)),
            pl.BlockSpec((1, kdim), lambda i: (0, 0)),
            pl.BlockSpec((kdim, ow), lambda i: (0, 0)),
        ],
        out_specs=[
            pl.BlockSpec((_BM, ow), lambda i: (i, 0)),
            pl.BlockSpec((_BM, n), lambda i: (i, 0)),
        ],
        out_shape=[
            jax.ShapeDtypeStruct((n, ow), jnp.bfloat16),
            jax.ShapeDtypeStruct((n, n), jnp.int8),
        ],
        scratch_shapes=[pltpu.VMEM((n, kdim), jnp.bfloat16)],
        compiler_params=pltpu.CompilerParams(
            dimension_semantics=("arbitrary",)),
    )(x_in, W1, adj, b, wn)


_BM2 = 1000


_NKC = 4


def _layer_body(adj_ref, y_ref, b_ref, wn_ref, bn_ref, o_ref, *, out_f32):
    n = adj_ref.shape[1]
    ck = n // _NKC
    acc = None
    for k in range(_NKC):
        a_k = adj_ref[:, pl.ds(k * ck, ck)].astype(jnp.bfloat16)
        part = jnp.dot(a_k, y_ref[pl.ds(k * ck, ck), :],
                       preferred_element_type=jnp.float32)
        acc = part if acc is None else acc + part
    h = jnp.maximum(acc + b_ref[...], 0.0).astype(jnp.bfloat16)
    r = jnp.dot(h, wn_ref[...], preferred_element_type=jnp.float32)
    if out_f32:
        o_ref[...] = r + bn_ref[...]
    else:
        o_ref[...] = (r * (1.0 / 127.0)).astype(jnp.bfloat16)


def _fused_layer(adj_q, y, b, wn, bn, out_f32):
    n = adj_q.shape[0]
    kdim = y.shape[1]
    ow = wn.shape[1]
    return pl.pallas_call(
        functools.partial(_layer_body, out_f32=out_f32),
        grid=(n // _BM2,),
        in_specs=[
            pl.BlockSpec((_BM2, n), lambda i: (i, 0)),
            pl.BlockSpec((n, kdim), lambda i: (0, 0)),
            pl.BlockSpec((1, kdim), lambda i: (0, 0)),
            pl.BlockSpec((kdim, ow), lambda i: (0, 0)),
            pl.BlockSpec((1, ow), lambda i: (0, 0)),
        ],
        out_specs=pl.BlockSpec((_BM2, ow), lambda i: (i, 0)),
        out_shape=jax.ShapeDtypeStruct(
            (n, ow), jnp.float32 if out_f32 else jnp.bfloat16),
        compiler_params=pltpu.CompilerParams(
            dimension_semantics=("parallel",)),
    )(adj_q, y, b, wn, bn)


def _make_seg_sum():
    mesh = plsc.VectorSubcoreMesh(core_axis_name="c", subcore_axis_name="s")

    @functools.partial(
        pl.kernel,
        mesh=mesh,
        out_type=jax.ShapeDtypeStruct((_TILES, _N_SEG, _N_SEG), jnp.float32),
        scratch_types=[
            pltpu.VMEM((_ROWS_PER_TILE, _N_SEG), jnp.float32),
            pltpu.VMEM((_CPT, _CHUNK), jnp.int32),
            pltpu.VMEM((_N_SEG, _N_SEG), jnp.float32),
        ],
    )
    def seg_sum(x_hbm, idx_hbm, out_hbm, rows_v, idx_v, acc_v):
        cid = lax.axis_index("c")
        sid = lax.axis_index("s")
        wid = sid * 2 + cid
        pltpu.sync_copy(
            x_hbm.at[pl.ds(wid * _ROWS_PER_TILE, _ROWS_PER_TILE)], rows_v)
        pltpu.sync_copy(idx_hbm.at[wid], idx_v)

        zero = jnp.zeros((16,), jnp.float32)
        for r in range(_N_SEG):
            for j in range(_N_SEG // 16):
                acc_v[r, pl.ds(j * 16, 16)] = zero

        for c in range(_CPT):
            for g in range(_CHUNK // 16):
                svec = idx_v[c, pl.ds(g * 16, 16)]
                for k in range(16):
                    s = svec[k]
                    r = c * _CHUNK + g * 16 + k
                    for j in range(_N_SEG // 16):
                        plsc.addupdate(acc_v.at[s, pl.ds(j * 16, 16)],
                                       rows_v[r, pl.ds(j * 16, 16)])

        pltpu.sync_copy(acc_v, out_hbm.at[wid])

    return seg_sum


_seg_sum = _make_seg_sum()


def _tail_body(p_ref, o_ref):
    p = jnp.sum(p_ref[...], axis=0)
    m = jnp.max(p, axis=1, keepdims=True)
    s = jnp.sum(jnp.exp(p - m), axis=1, keepdims=True)
    o_ref[...] = (p - m) - jnp.log(s)


def _tail(parts):
    return pl.pallas_call(
        _tail_body,
        in_specs=[pl.BlockSpec((_TILES, _N_SEG, _N_SEG),
                               lambda: (0, 0, 0))],
        out_specs=pl.BlockSpec((_N_SEG, _N_SEG), lambda: (0, 0)),
        out_shape=jax.ShapeDtypeStruct((_N_SEG, _N_SEG), jnp.float32),
    )(parts)


def kernel(x_in, adj, idx, W1, b1, W2, b2, W3, b3, W4, b4):
    bf16 = jnp.bfloat16
    y2, adj_q = _layer1(x_in, W1, adj, b1.reshape(1, -1), W2.astype(bf16))
    y3 = _fused_layer(adj_q, y2, b2.reshape(1, -1), W3.astype(bf16),
                      jnp.zeros((1, W3.shape[1]), jnp.float32), False)
    x4 = _fused_layer(adj_q, y3, b3.reshape(1, -1), W4.astype(bf16),
                      b4.reshape(1, -1), True)

    x4p = jnp.pad(x4, ((0, _N_PAD - _N), (0, 0)))
    idxp = jnp.pad(idx.astype(jnp.int32), (0, _N_PAD - _N))
    idxp = idxp.reshape(_TILES, _CPT, _CHUNK)
    parts = _seg_sum(x4p, idxp)
    return _tail(parts)
